# baseline (speedup 1.0000x reference)
import functools

import jax
import jax.numpy as jnp
from jax import lax
from jax.experimental import pallas as pl
from jax.experimental.pallas import tpu as pltpu
from jax.experimental.pallas import tpu_sc as plsc

D_MODEL = 64
L_SEQ = 1024
CHUNK = 128
N_CHUNK = L_SEQ // CHUNK
HALF = L_SEQ // 2
NBUF = 3


def _tc_combine(tok_ref, pos_ref, out_ref):
    out_ref[...] = tok_ref[...][:, None, :] + pos_ref[...][None, :, :]


def _sc_body(rows_per_w, num_cores, n_tab, tokens_hbm, comb_hbm, out_hbm,
             tok_v, idx_v, bufs, gsems, osems):
    wid = lax.axis_index("s") * num_cores + lax.axis_index("c")
    base = wid * rows_per_w

    pltpu.sync_copy(tokens_hbm.at[pl.ds(base, rows_per_w)], tok_v)

    def do_row(r, _):
        carry = jnp.int32(0)
        for j in range(N_CHUNK):
            for c in range(CHUNK // 16):
                t = tok_v[r, pl.ds(j * CHUNK + c * 16, 16)]
                m = jnp.minimum(t, 1)
                cs = plsc.cumsum(m)
                idx_v[r, j, pl.ds(c * 16, 16)] = t * n_tab + (cs + carry) * m
                carry = carry + jnp.sum(m)
        return 0

    lax.fori_loop(0, rows_per_w, do_row, 0)

    n_units = rows_per_w * 2
    gcps = [None] * n_units
    ocps = [None] * n_units

    def fire_gathers(u):
        b = u % NBUF
        r, h = u // 2, u % 2
        return [pltpu.async_copy(
                    comb_hbm.at[idx_v.at[r, h * (N_CHUNK // 2) + j]],
                    bufs[b].at[pl.ds(j * CHUNK, CHUNK)], gsems[b])
                for j in range(N_CHUNK // 2)]

    def fire_out(u):
        b = u % NBUF
        r, h = u // 2, u % 2
        return pltpu.async_copy(
            bufs[b], out_hbm.at[base + r, pl.ds(h * HALF, HALF)], osems[b])

    for u in range(n_units):
        if u >= NBUF:
            ocps[u - NBUF].wait()
        gcps[u] = fire_gathers(u)
        if u >= 1:
            for cp in gcps[u - 1]:
                cp.wait()
            ocps[u - 1] = fire_out(u - 1)
    for cp in gcps[n_units - 1]:
        cp.wait()
    ocps[n_units - 1] = fire_out(n_units - 1)
    ocps[n_units - 2].wait()
    ocps[n_units - 1].wait()


def kernel(tokens, tok_emb, pos_emb):
    B, K, L = tokens.shape
    assert L == L_SEQ and tok_emb.shape[1] == D_MODEL
    R = B * K
    n_vocab = tok_emb.shape[0]
    n_tab = pos_emb.shape[0]

    comb = pl.pallas_call(
        _tc_combine,
        out_shape=jax.ShapeDtypeStruct((n_vocab, n_tab, D_MODEL), jnp.float32),
    )(tok_emb, pos_emb).reshape(n_vocab * n_tab, D_MODEL)

    info = plsc.get_sparse_core_info()
    nw = info.num_cores * info.num_subcores
    rows_per_w = R // nw
    assert rows_per_w * nw == R

    tokens2d = tokens.reshape(R, L_SEQ).astype(jnp.int32)

    mesh = plsc.VectorSubcoreMesh(core_axis_name="c", subcore_axis_name="s")
    run = pl.kernel(
        functools.partial(_sc_body, rows_per_w, info.num_cores, n_tab),
        out_type=jax.ShapeDtypeStruct((R, L_SEQ, D_MODEL), jnp.float32),
        mesh=mesh,
        scratch_types=[
            pltpu.VMEM((rows_per_w, L_SEQ), jnp.int32),
            pltpu.VMEM((rows_per_w, N_CHUNK, CHUNK), jnp.int32),
            [pltpu.VMEM((HALF, D_MODEL), jnp.float32)] * NBUF,
            [pltpu.SemaphoreType.DMA] * NBUF,
            [pltpu.SemaphoreType.DMA] * NBUF,
        ],
        compiler_params=pltpu.CompilerParams(use_tc_tiling_on_sc=False,
                                             needs_layout_passes=False),
    )
    out = run(tokens2d, comb)
    return out.reshape(B, K, L, D_MODEL)


# speedup vs baseline: 2.4422x; 2.4422x over previous
import functools

import jax
import jax.numpy as jnp
from jax import lax
from jax.experimental import pallas as pl
from jax.experimental.pallas import tpu as pltpu
from jax.experimental.pallas import tpu_sc as plsc

D_MODEL = 64
L_SEQ = 1024
CHUNK = 128
N_CHUNK = L_SEQ // CHUNK
HALF = L_SEQ // 2
NBUF = 3


def _tc_combine(tok_ref, pos_ref, out_ref):
    pos = pos_ref[...]
    for t in range(tok_ref.shape[0]):
        out_ref[pl.ds(t * pos.shape[0], pos.shape[0]), :] = \
            pos + tok_ref[pl.ds(t, 1), :]


def _sc_body(rows_per_w, num_cores, num_subcores, n_tab, n_comb,
             tokens_hbm, comb_hbm, out_hbm,
             tok_v, idx_v, bufs, comb_sh, stage_sem, gsems, osems):
    cid = lax.axis_index("c")
    sid = lax.axis_index("s")
    wid = sid * num_cores + cid
    base = wid * rows_per_w

    slice_rows = n_comb // num_subcores
    tail = n_comb - slice_rows * num_subcores
    lo = sid * slice_rows
    cp_stage = pltpu.async_copy(
        comb_hbm.at[pl.ds(lo, slice_rows)], comb_sh.at[pl.ds(lo, slice_rows)],
        stage_sem)
    @pl.when(sid == 0)
    def _():
        pltpu.sync_copy(comb_hbm.at[pl.ds(n_comb - tail, tail)],
                        comb_sh.at[pl.ds(n_comb - tail, tail)])

    pltpu.sync_copy(tokens_hbm.at[pl.ds(base, rows_per_w)], tok_v)

    def compute_row_idx(r):
        def chunk(i, carry):
            t = tok_v[r, pl.ds(i * 16, 16)]
            m = jnp.minimum(t, 1)
            cs = plsc.cumsum(m)
            idx_v[r, i // 8, pl.ds((i % 8) * 16, 16)] = \
                t * n_tab + (cs + carry) * m
            return carry + jnp.sum(m)
        lax.fori_loop(0, L_SEQ // 16, chunk, jnp.int32(0))

    compute_row_idx(0)
    cp_stage.wait()
    plsc.subcore_barrier()

    n_units = rows_per_w * N_CHUNK
    gcps = [None] * n_units
    ocps = [None] * n_units

    def fire_gather(u):
        b = u % NBUF
        r, j = u // N_CHUNK, u % N_CHUNK
        return pltpu.async_copy(comb_sh.at[idx_v.at[r, j]], bufs[b], gsems[b])

    def fire_out(u):
        b = u % NBUF
        r, j = u // N_CHUNK, u % N_CHUNK
        return pltpu.async_copy(
            bufs[b], out_hbm.at[base + r, pl.ds(j * CHUNK, CHUNK)], osems[b])

    for u in range(n_units):
        if u >= NBUF:
            ocps[u - NBUF].wait()
        gcps[u] = fire_gather(u)
        if u % N_CHUNK == N_CHUNK - 1 and u // N_CHUNK + 1 < rows_per_w:
            compute_row_idx(u // N_CHUNK + 1)
        if u >= 1:
            gcps[u - 1].wait()
            ocps[u - 1] = fire_out(u - 1)
    gcps[n_units - 1].wait()
    ocps[n_units - 1] = fire_out(n_units - 1)
    ocps[n_units - 2].wait()
    ocps[n_units - 1].wait()


def kernel(tokens, tok_emb, pos_emb):
    B, K, L = tokens.shape
    assert L == L_SEQ and tok_emb.shape[1] == D_MODEL
    R = B * K
    n_vocab = tok_emb.shape[0]
    n_tab = pos_emb.shape[0]
    n_comb = n_vocab * n_tab

    comb = pl.pallas_call(
        _tc_combine,
        out_shape=jax.ShapeDtypeStruct((n_comb, D_MODEL), jnp.float32),
    )(tok_emb, pos_emb)

    info = plsc.get_sparse_core_info()
    nw = info.num_cores * info.num_subcores
    rows_per_w = R // nw
    assert rows_per_w * nw == R

    tokens2d = tokens.reshape(R, L_SEQ).astype(jnp.int32)

    mesh = plsc.VectorSubcoreMesh(core_axis_name="c", subcore_axis_name="s")
    run = pl.kernel(
        functools.partial(_sc_body, rows_per_w, info.num_cores,
                          info.num_subcores, n_tab, n_comb),
        out_type=jax.ShapeDtypeStruct((R, L_SEQ, D_MODEL), jnp.float32),
        mesh=mesh,
        scratch_types=[
            pltpu.VMEM((rows_per_w, L_SEQ), jnp.int32),
            pltpu.VMEM((rows_per_w, N_CHUNK, CHUNK), jnp.int32),
            [pltpu.VMEM((CHUNK, D_MODEL), jnp.float32)] * NBUF,
            pltpu.VMEM_SHARED((n_comb, D_MODEL), jnp.float32),
            pltpu.SemaphoreType.DMA,
            [pltpu.SemaphoreType.DMA] * NBUF,
            [pltpu.SemaphoreType.DMA] * NBUF,
        ],
        compiler_params=pltpu.CompilerParams(use_tc_tiling_on_sc=False,
                                             needs_layout_passes=False),
    )
    out = run(tokens2d, comb)
    return out.reshape(B, K, L, D_MODEL)
